# baseline (speedup 1.0000x reference)
import jax
import jax.numpy as jnp
from jax.experimental import pallas as pl
from jax.experimental.pallas import tpu as pltpu


_BATCH_TILE = 512
_LOAD_STREAMS = 4


def _gather_kernel(cls_smem, emb_any, o_ref, emb_ref, sems):
    tb = o_ref.shape[0]
    base = pl.program_id(0) * tb
    n_class = emb_any.shape[0]
    chunk = n_class // _LOAD_STREAMS

    def stream_copy(q):
        rows_n = chunk if q < _LOAD_STREAMS - 1 else n_class - chunk * q
        return pltpu.make_async_copy(
            emb_any.at[pl.ds(q * chunk, rows_n)],
            emb_ref.at[pl.ds(q * chunk, rows_n)], sems.at[q])

    @pl.when(pl.program_id(0) % (pl.num_programs(0) // 2) == 0)
    def _():
        for q in range(_LOAD_STREAMS):
            stream_copy(q).start()
        for q in range(_LOAD_STREAMS):
            stream_copy(q).wait()
    for g in range(tb // 8):
        rows = []
        for j in range(8):
            idx = cls_smem[base + g * 8 + j]
            rows.append(emb_ref[idx, :])
        o_ref[pl.ds(g * 8, 8), :] = jnp.stack(rows, axis=0)


def kernel(cls, cls_emb):
    cls_shape = cls.shape
    batch = 1
    for d in cls_shape:
        batch *= d
    n_class, cond_dim = cls_emb.shape
    out_dtype = cls_emb.dtype

    cls_i32 = jnp.clip(cls.reshape(batch).astype(jnp.int32), 0, n_class - 1)

    tb = min(_BATCH_TILE, batch)
    grain = 2 * tb
    padded_batch = ((batch + grain - 1) // grain) * grain
    if padded_batch != batch:
        cls_i32 = jnp.pad(cls_i32, (0, padded_batch - batch))

    itemsize = jnp.dtype(out_dtype).itemsize
    vmem_limit = min(
        n_class * cond_dim * itemsize
        + 4 * tb * cond_dim * itemsize
        + 4 * 1024 * 1024,
        64 * 1024 * 1024,
    )

    out = pl.pallas_call(
        _gather_kernel,
        out_shape=jax.ShapeDtypeStruct((padded_batch, cond_dim), out_dtype),
        grid_spec=pltpu.PrefetchScalarGridSpec(
            num_scalar_prefetch=1,
            grid=(padded_batch // tb,),
            in_specs=[pl.BlockSpec(memory_space=pl.ANY)],
            out_specs=pl.BlockSpec((tb, cond_dim), lambda i, s: (i, 0)),
            scratch_shapes=[
                pltpu.VMEM((n_class, cond_dim), jnp.float32),
                pltpu.SemaphoreType.DMA((_LOAD_STREAMS,)),
            ],
        ),
        compiler_params=pltpu.CompilerParams(
            dimension_semantics=("parallel",),
            vmem_limit_bytes=int(vmem_limit)),
    )(cls_i32, cls_emb)

    if padded_batch != batch:
        out = out[:batch]
    return out.reshape(*cls_shape, cond_dim)


# speedup vs baseline: 1.1390x; 1.1390x over previous
import jax
import jax.numpy as jnp
from jax.experimental import pallas as pl
from jax.experimental.pallas import tpu as pltpu


_BATCH_TILE = 512


def _gather_kernel(cls_smem, emb_ref, o_ref):
    tb = o_ref.shape[0]
    base = pl.program_id(0) * tb
    for g in range(tb // 8):
        rows = []
        for j in range(8):
            idx = cls_smem[base + g * 8 + j]
            rows.append(emb_ref[idx, :])
        o_ref[pl.ds(g * 8, 8), :] = jnp.stack(rows, axis=0)


def kernel(cls, cls_emb):
    cls_shape = cls.shape
    batch = 1
    for d in cls_shape:
        batch *= d
    n_class, cond_dim = cls_emb.shape
    out_dtype = cls_emb.dtype

    cls_i32 = jnp.clip(cls.reshape(batch).astype(jnp.int32), 0, n_class - 1)

    tb = min(_BATCH_TILE, batch)
    grain = 2 * tb
    padded_batch = ((batch + grain - 1) // grain) * grain
    if padded_batch != batch:
        cls_i32 = jnp.pad(cls_i32, (0, padded_batch - batch))

    itemsize = jnp.dtype(out_dtype).itemsize
    vmem_limit = min(
        n_class * cond_dim * itemsize
        + 4 * tb * cond_dim * itemsize
        + 4 * 1024 * 1024,
        64 * 1024 * 1024,
    )

    out = pl.pallas_call(
        _gather_kernel,
        out_shape=jax.ShapeDtypeStruct((padded_batch, cond_dim), out_dtype),
        grid_spec=pltpu.PrefetchScalarGridSpec(
            num_scalar_prefetch=1,
            grid=(padded_batch // tb,),
            in_specs=[
                pl.BlockSpec((n_class, cond_dim), lambda i, s: (0, 0),
                             pipeline_mode=pl.Buffered(1)),
            ],
            out_specs=pl.BlockSpec((tb, cond_dim), lambda i, s: (i, 0)),
        ),
        compiler_params=pltpu.CompilerParams(
            dimension_semantics=("parallel",),
            vmem_limit_bytes=int(vmem_limit)),
    )(cls_i32, cls_emb)

    if padded_batch != batch:
        out = out[:batch]
    return out.reshape(*cls_shape, cond_dim)
